# baseline (speedup 1.0000x reference)
import jax
import jax.numpy as jnp
from jax import lax
from jax.experimental import pallas as pl
from jax.experimental.pallas import tpu as pltpu
from jax.experimental.pallas import tpu_sc as plsc

N = 10000
D = 128
HH = 256
NT = 2
E = 320000
HALF = D // 2
TILES = 16
PER_TILE = E // TILES
K = 80
NB = 252
EPT_PAD = NB * K
SEG = 20480
ROWS_PER_TILE = SEG // TILES
BN = 1000
GRID = N // BN


def _edge_mlp_body(nf_ref, w1_ref, b1_ref, w2_ref, b2_ref, out_ref):
    x = nf_ref[...]
    for t in range(NT):
        h = jnp.maximum(
            jnp.dot(x, w1_ref[t], preferred_element_type=jnp.float32) + b1_ref[t],
            0.0)
        m = jnp.maximum(
            jnp.dot(h, w2_ref[t], preferred_element_type=jnp.float32) + b2_ref[t],
            0.0)
        out_ref[t] = m


def _edge_mlp(nf, w1s, b1s, w2s, b2s, interpret=False):
    return pl.pallas_call(
        _edge_mlp_body,
        grid=(GRID,),
        in_specs=[
            pl.BlockSpec((BN, D), lambda j: (j, 0)),
            pl.BlockSpec((NT, D, HH), lambda j: (0, 0, 0)),
            pl.BlockSpec((NT, 1, HH), lambda j: (0, 0, 0)),
            pl.BlockSpec((NT, HH, D), lambda j: (0, 0, 0)),
            pl.BlockSpec((NT, 1, D), lambda j: (0, 0, 0)),
        ],
        out_specs=pl.BlockSpec((NT, BN, D), lambda j: (0, j, 0)),
        out_shape=jax.ShapeDtypeStruct((NT, N, D), jnp.float32),
        interpret=interpret,
    )(nf, w1s, b1s, w2s, b2s)


def _node_mlp_body(nf_ref, a00_ref, a01_ref, a10_ref, a11_ref,
                   w1_ref, b1_ref, w2_ref, b2_ref, out_ref):
    x = jnp.maximum(nf_ref[...], 0.0)
    h = jnp.dot(x, w1_ref[0:D], preferred_element_type=jnp.float32)
    h += jnp.dot(a00_ref[0], w1_ref[D:D + HALF],
                 preferred_element_type=jnp.float32)
    h += jnp.dot(a01_ref[0], w1_ref[D + HALF:2 * D],
                 preferred_element_type=jnp.float32)
    h += jnp.dot(a10_ref[0], w1_ref[2 * D:2 * D + HALF],
                 preferred_element_type=jnp.float32)
    h += jnp.dot(a11_ref[0], w1_ref[2 * D + HALF:3 * D],
                 preferred_element_type=jnp.float32)
    h = jnp.maximum(h + b1_ref[...], 0.0)
    out_ref[...] = (jnp.dot(h, w2_ref[...], preferred_element_type=jnp.float32)
                    + b2_ref[...])


def _node_mlp(nf, agg, w1, b1, w2, b2, interpret=False):
    return pl.pallas_call(
        _node_mlp_body,
        grid=(GRID,),
        in_specs=[
            pl.BlockSpec((BN, D), lambda j: (j, 0)),
            pl.BlockSpec((1, BN, HALF), lambda j: (0, j, 0)),
            pl.BlockSpec((1, BN, HALF), lambda j: (1, j, 0)),
            pl.BlockSpec((1, BN, HALF), lambda j: (0, GRID + j, 0)),
            pl.BlockSpec((1, BN, HALF), lambda j: (1, GRID + j, 0)),
            pl.BlockSpec((3 * D, HH), lambda j: (0, 0)),
            pl.BlockSpec((1, HH), lambda j: (0, 0)),
            pl.BlockSpec((HH, D), lambda j: (0, 0)),
            pl.BlockSpec((1, D), lambda j: (0, 0)),
        ],
        out_specs=pl.BlockSpec((BN, D), lambda j: (j, 0)),
        out_shape=jax.ShapeDtypeStruct((N, D), jnp.float32),
        interpret=interpret,
    )(nf, agg, agg, agg, agg, w1, b1, w2, b2)


def _sc_agg_body(m4_hbm, gidx_hbm, sidx_hbm, zeros_hbm, out_hbm,
                 agg_sh, gv, sv, rows, semi, semg, sems):
    c = lax.axis_index("c")
    s = lax.axis_index("s")
    pltpu.sync_copy(zeros_hbm, agg_sh.at[pl.ds(s * ROWS_PER_TILE, ROWS_PER_TILE)])
    plsc.subcore_barrier()

    def fetch(b, j):
        pltpu.async_copy(gidx_hbm.at[c, s, b], gv[j], semi[j])
        pltpu.async_copy(sidx_hbm.at[s, b], sv[j], semi[j])

    def wait_fetch(b, j):
        pltpu.make_async_copy(gidx_hbm.at[c, s, b], gv[j], semi[j]).wait()
        pltpu.make_async_copy(sidx_hbm.at[s, b], sv[j], semi[j]).wait()

    def wait_scatter(j):
        pltpu.make_async_copy(rows[j], agg_sh.at[sv[j]], sems[j]).wait()

    fetch(0, 0)
    fetch(1, 1)
    wait_fetch(0, 0)
    pltpu.async_copy(m4_hbm.at[gv[0]], rows[0], semg[0])

    def maybe_when(cond, fn):
        if isinstance(cond, bool):
            if cond:
                fn()
        else:
            pl.when(cond)(fn)

    def step(b, j, drain):
        j1 = (j + 1) % 4
        j2 = (j + 2) % 4

        def prefetch_next():
            wait_fetch(b + 1, j1)
            pltpu.async_copy(m4_hbm.at[gv[j1]], rows[j1], semg[j1])

        maybe_when(b + 1 < NB, prefetch_next)

        pltpu.make_async_copy(m4_hbm.at[gv[j]], rows[j], semg[j]).wait()
        pltpu.async_copy(rows[j], agg_sh.at[sv[j]], sems[j], add=True)

        if drain:
            wait_scatter(j2)

        maybe_when(b + 2 < NB, lambda: fetch(b + 2, j2))

    def body(i, carry):
        b0 = 4 * i
        for u in range(4):
            step(b0 + u, u, True)
        return carry

    step(0, 0, False)
    step(1, 1, False)
    step(2, 2, True)
    step(3, 3, True)
    lax.fori_loop(1, NB // 4, body, 0)
    wait_scatter((NB - 2) % 4)
    wait_scatter((NB - 1) % 4)
    plsc.subcore_barrier()
    pltpu.sync_copy(agg_sh.at[pl.ds(s * ROWS_PER_TILE, ROWS_PER_TILE)],
                    out_hbm.at[c, pl.ds(s * ROWS_PER_TILE, ROWS_PER_TILE)])


def _sc_agg(m4, gidx, sidx, zeros):
    mesh = plsc.VectorSubcoreMesh(core_axis_name="c", subcore_axis_name="s")
    kern = pl.kernel(
        _sc_agg_body,
        out_type=jax.ShapeDtypeStruct((NT, SEG, HALF), jnp.float32),
        mesh=mesh,
        scratch_types=[
            pltpu.VMEM_SHARED((SEG, HALF), jnp.float32),
            [pltpu.VMEM((K,), jnp.int32) for _ in range(4)],
            [pltpu.VMEM((K,), jnp.int32) for _ in range(4)],
            [pltpu.VMEM((K, HALF), jnp.float32) for _ in range(4)],
            [pltpu.SemaphoreType.DMA for _ in range(4)],
            [pltpu.SemaphoreType.DMA for _ in range(4)],
            [pltpu.SemaphoreType.DMA for _ in range(4)],
        ],
        compiler_params=pltpu.CompilerParams(use_tc_tiling_on_sc=False),
    )
    return kern(m4, gidx, sidx, zeros)


def kernel(x, edge_index, edge_type, params):
    src = edge_index[0].astype(jnp.int32)
    dst = edge_index[1].astype(jnp.int32)
    et = edge_type.astype(jnp.int32)
    base = et * N + src
    pad = EPT_PAD - PER_TILE
    gidx = jnp.pad(
        jnp.stack([base * 2, base * 2 + 1]).reshape(NT, TILES, PER_TILE),
        ((0, 0), (0, 0), (0, pad))).reshape(NT, TILES, NB, K)
    sidx = jnp.pad((et * N + dst).reshape(TILES, PER_TILE),
                   ((0, 0), (0, pad)),
                   constant_values=NT * N).reshape(TILES, NB, K)
    zeros = jnp.zeros((ROWS_PER_TILE, HALF), jnp.float32)

    nf = x
    for lp in params:
        w1s = jnp.stack([lp["edge0"]["W1"], lp["edge1"]["W1"]])
        b1s = jnp.stack([lp["edge0"]["b1"], lp["edge1"]["b1"]])[:, None, :]
        w2s = jnp.stack([lp["edge0"]["W2"], lp["edge1"]["W2"]])
        b2s = jnp.stack([lp["edge0"]["b2"], lp["edge1"]["b2"]])[:, None, :]
        m = _edge_mlp(nf, w1s, b1s, w2s, b2s)
        m4 = m.reshape(2 * NT * N, HALF)
        agg = _sc_agg(m4, gidx, sidx, zeros)
        npar = lp["node"]
        nf = _node_mlp(nf, agg, npar["W1"], npar["b1"][None, :],
                       npar["W2"], npar["b2"][None, :])
    return nf


# speedup vs baseline: 1.2091x; 1.2091x over previous
import jax
import jax.numpy as jnp
from jax import lax
from jax.experimental import pallas as pl
from jax.experimental.pallas import tpu as pltpu
from jax.experimental.pallas import tpu_sc as plsc

N = 10000
D = 128
HH = 256
NT = 2
E = 320000
HALF = D // 2
TILES = 16
PER_TILE = E // TILES
K = 80
NB = 250
EPT_PAD = NB * K
SEG = 20480
ROWS_PER_TILE = SEG // TILES
BN = 1000
GRID = N // BN


def _edge_mlp_body(nf_ref, w1_ref, b1_ref, w2_ref, b2_ref, out_ref):
    x = nf_ref[...]
    for t in range(NT):
        h = jnp.maximum(
            jnp.dot(x, w1_ref[t], preferred_element_type=jnp.float32) + b1_ref[t],
            0.0)
        m = jnp.maximum(
            jnp.dot(h, w2_ref[t], preferred_element_type=jnp.float32) + b2_ref[t],
            0.0)
        out_ref[t] = m


def _edge_mlp(nf, w1s, b1s, w2s, b2s, interpret=False):
    return pl.pallas_call(
        _edge_mlp_body,
        grid=(GRID,),
        in_specs=[
            pl.BlockSpec((BN, D), lambda j: (j, 0)),
            pl.BlockSpec((NT, D, HH), lambda j: (0, 0, 0)),
            pl.BlockSpec((NT, 1, HH), lambda j: (0, 0, 0)),
            pl.BlockSpec((NT, HH, D), lambda j: (0, 0, 0)),
            pl.BlockSpec((NT, 1, D), lambda j: (0, 0, 0)),
        ],
        out_specs=pl.BlockSpec((NT, BN, D), lambda j: (0, j, 0)),
        out_shape=jax.ShapeDtypeStruct((NT, N, D), jnp.float32),
        interpret=interpret,
    )(nf, w1s, b1s, w2s, b2s)


def _node_mlp_body(nf_ref, a00_ref, a01_ref, a10_ref, a11_ref,
                   w1_ref, b1_ref, w2_ref, b2_ref, out_ref):
    x = jnp.maximum(nf_ref[...], 0.0)
    h = jnp.dot(x, w1_ref[0:D], preferred_element_type=jnp.float32)
    h += jnp.dot(a00_ref[0], w1_ref[D:D + HALF],
                 preferred_element_type=jnp.float32)
    h += jnp.dot(a01_ref[0], w1_ref[D + HALF:2 * D],
                 preferred_element_type=jnp.float32)
    h += jnp.dot(a10_ref[0], w1_ref[2 * D:2 * D + HALF],
                 preferred_element_type=jnp.float32)
    h += jnp.dot(a11_ref[0], w1_ref[2 * D + HALF:3 * D],
                 preferred_element_type=jnp.float32)
    h = jnp.maximum(h + b1_ref[...], 0.0)
    out_ref[...] = (jnp.dot(h, w2_ref[...], preferred_element_type=jnp.float32)
                    + b2_ref[...])


def _node_mlp(nf, agg, w1, b1, w2, b2, interpret=False):
    return pl.pallas_call(
        _node_mlp_body,
        grid=(GRID,),
        in_specs=[
            pl.BlockSpec((BN, D), lambda j: (j, 0)),
            pl.BlockSpec((1, BN, HALF), lambda j: (0, j, 0)),
            pl.BlockSpec((1, BN, HALF), lambda j: (1, j, 0)),
            pl.BlockSpec((1, BN, HALF), lambda j: (0, GRID + j, 0)),
            pl.BlockSpec((1, BN, HALF), lambda j: (1, GRID + j, 0)),
            pl.BlockSpec((3 * D, HH), lambda j: (0, 0)),
            pl.BlockSpec((1, HH), lambda j: (0, 0)),
            pl.BlockSpec((HH, D), lambda j: (0, 0)),
            pl.BlockSpec((1, D), lambda j: (0, 0)),
        ],
        out_specs=pl.BlockSpec((BN, D), lambda j: (j, 0)),
        out_shape=jax.ShapeDtypeStruct((N, D), jnp.float32),
        interpret=interpret,
    )(nf, agg, agg, agg, agg, w1, b1, w2, b2)


def _sc_agg_body(m4_hbm, gidx_hbm, sidx_hbm, zeros_hbm, out_hbm,
                 agg_sh, gv, sv, rows, semi, semg):
    c = lax.axis_index("c")
    s = lax.axis_index("s")
    pltpu.sync_copy(zeros_hbm, agg_sh.at[pl.ds(s * ROWS_PER_TILE, ROWS_PER_TILE)])
    plsc.subcore_barrier()

    def fetch(b, j):
        pltpu.async_copy(gidx_hbm.at[c, s, b], gv[j], semi[j])
        pltpu.async_copy(sidx_hbm.at[s, b], sv[j], semi[j])

    def wait_fetch(b, j):
        pltpu.make_async_copy(gidx_hbm.at[c, s, b], gv[j], semi[j]).wait()
        pltpu.make_async_copy(sidx_hbm.at[s, b], sv[j], semi[j]).wait()

    fetch(0, 0)
    fetch(1, 1)
    wait_fetch(0, 0)
    pltpu.async_copy(m4_hbm.at[gv[0]], rows[0], semg[0])

    def step(b, j):
        j1 = (j + 1) % 2

        @pl.when(b + 1 < NB)
        def _():
            wait_fetch(b + 1, j1)
            pltpu.async_copy(m4_hbm.at[gv[j1]], rows[j1], semg[j1])

        pltpu.make_async_copy(m4_hbm.at[gv[j]], rows[j], semg[j]).wait()
        pltpu.sync_copy(rows[j], agg_sh.at[sv[j]], add=True)

        @pl.when(b + 2 < NB)
        def _():
            fetch(b + 2, j)

    def body(i, carry):
        b0 = 2 * i
        step(b0, 0)
        step(b0 + 1, 1)
        return carry

    lax.fori_loop(0, NB // 2, body, 0)
    plsc.subcore_barrier()
    pltpu.sync_copy(agg_sh.at[pl.ds(s * ROWS_PER_TILE, ROWS_PER_TILE)],
                    out_hbm.at[c, pl.ds(s * ROWS_PER_TILE, ROWS_PER_TILE)])


def _sc_agg(m4, gidx, sidx, zeros):
    mesh = plsc.VectorSubcoreMesh(core_axis_name="c", subcore_axis_name="s")
    kern = pl.kernel(
        _sc_agg_body,
        out_type=jax.ShapeDtypeStruct((NT, SEG, HALF), jnp.float32),
        mesh=mesh,
        scratch_types=[
            pltpu.VMEM_SHARED((SEG, HALF), jnp.float32),
            [pltpu.VMEM((K,), jnp.int32) for _ in range(2)],
            [pltpu.VMEM((K,), jnp.int32) for _ in range(2)],
            [pltpu.VMEM((K, HALF), jnp.float32) for _ in range(2)],
            [pltpu.SemaphoreType.DMA for _ in range(2)],
            [pltpu.SemaphoreType.DMA for _ in range(2)],
        ],
        compiler_params=pltpu.CompilerParams(use_tc_tiling_on_sc=False),
    )
    return kern(m4, gidx, sidx, zeros)


def kernel(x, edge_index, edge_type, params):
    src = edge_index[0].astype(jnp.int32)
    dst = edge_index[1].astype(jnp.int32)
    et = edge_type.astype(jnp.int32)
    base = et * N + src
    pad = EPT_PAD - PER_TILE
    gidx = jnp.pad(
        jnp.stack([base * 2, base * 2 + 1]).reshape(NT, TILES, PER_TILE),
        ((0, 0), (0, 0), (0, pad))).reshape(NT, TILES, NB, K)
    sidx = jnp.pad((et * N + dst).reshape(TILES, PER_TILE),
                   ((0, 0), (0, pad)),
                   constant_values=NT * N).reshape(TILES, NB, K)
    zeros = jnp.zeros((ROWS_PER_TILE, HALF), jnp.float32)

    nf = x
    for lp in params:
        w1s = jnp.stack([lp["edge0"]["W1"], lp["edge1"]["W1"]])
        b1s = jnp.stack([lp["edge0"]["b1"], lp["edge1"]["b1"]])[:, None, :]
        w2s = jnp.stack([lp["edge0"]["W2"], lp["edge1"]["W2"]])
        b2s = jnp.stack([lp["edge0"]["b2"], lp["edge1"]["b2"]])[:, None, :]
        m = _edge_mlp(nf, w1s, b1s, w2s, b2s)
        m4 = m.reshape(2 * NT * N, HALF)
        agg = _sc_agg(m4, gidx, sidx, zeros)
        npar = lp["node"]
        nf = _node_mlp(nf, agg, npar["W1"], npar["b1"][None, :],
                       npar["W2"], npar["b2"][None, :])
    return nf
